# Initial kernel scaffold; baseline (speedup 1.0000x reference)
import jax
import jax.numpy as jnp
from jax.experimental import pallas as pl


def kernel(Z, table):
    raise NotImplementedError("write your pallas kernel here")


# speedup vs baseline: 1.3514x; 1.3514x over previous
import functools

import jax
import jax.numpy as jnp
from jax import lax
from jax.experimental import pallas as pl
from jax.experimental.pallas import tpu as pltpu
from jax.experimental.pallas import tpu_sc as plsc

N_ATOMS = 100000
EMB = 128
UNIT = 128
N_UNITS = (N_ATOMS + UNIT - 1) // UNIT
PAD_N = N_UNITS * UNIT
TAIL_ROWS = N_ATOMS - (N_UNITS - 1) * UNIT
NC, NS = 2, 16
NW = NC * NS
UNITS_PER_W = (N_UNITS + NW - 1) // NW


@functools.lru_cache(maxsize=None)
def _build():
    mesh = plsc.VectorSubcoreMesh(core_axis_name="c", subcore_axis_name="s")

    @functools.partial(
        pl.kernel,
        out_type=jax.ShapeDtypeStruct((N_ATOMS, EMB), jnp.float32),
        mesh=mesh,
        scratch_types=[
            pltpu.VMEM((UNIT,), jnp.int32),
            pltpu.VMEM((UNIT, EMB), jnp.float32),
            pltpu.SemaphoreType.DMA,
        ],
    )
    def emb(z_hbm, table_hbm, out_hbm, idx_v, rows_v, sem):
        wid = lax.axis_index("s") * NC + lax.axis_index("c")
        for u in range(UNITS_PER_W):
            unit = u * NW + wid

            @pl.when(unit < N_UNITS)
            def _():
                base = pl.multiple_of(unit * UNIT, UNIT)
                pltpu.sync_copy(z_hbm.at[pl.ds(base, UNIT)], idx_v)
                pltpu.async_copy(table_hbm.at[idx_v], rows_v, sem).wait()

                @pl.when(unit < N_UNITS - 1)
                def _full():
                    pltpu.sync_copy(rows_v, out_hbm.at[pl.ds(base, UNIT)])

                @pl.when(unit == N_UNITS - 1)
                def _tail():
                    pltpu.sync_copy(
                        rows_v.at[pl.ds(0, TAIL_ROWS)],
                        out_hbm.at[pl.ds(base, TAIL_ROWS)],
                    )

    return emb


def kernel(Z, table):
    z = jnp.pad(Z.astype(jnp.int32), (0, PAD_N - N_ATOMS))
    return _build()(z, table)
